# baseline (speedup 1.0000x reference)
import functools

import jax
import jax.numpy as jnp
from jax import lax
from jax.experimental import pallas as pl
from jax.experimental.pallas import tpu as pltpu
from jax.experimental.pallas import tpu_sc as plsc

N_NODES = 100000
N_EDGES = 6400000
NC = 2
NS = 16
NW = NC * NS
EPW = N_EDGES // NW
CHUNK = 4000
NCHUNKS = EPW // CHUNK
NGROUPS = NCHUNKS // 2
VPC = CHUNK // 16
UNROLL = 10
NPAD = 100352

_mesh = plsc.VectorSubcoreMesh(core_axis_name="c", subcore_axis_name="s")


@functools.partial(
    pl.kernel,
    out_type=jax.ShapeDtypeStruct((N_EDGES,), jnp.float32),
    mesh=_mesh,
    compiler_params=pltpu.CompilerParams(needs_layout_passes=False),
    scratch_types=[
        pltpu.VMEM((N_NODES,), jnp.float32),
        pltpu.VMEM((CHUNK,), jnp.int32),
        pltpu.VMEM((CHUNK,), jnp.int32),
        pltpu.VMEM((CHUNK,), jnp.float32),
        pltpu.VMEM((CHUNK,), jnp.float32),
        pltpu.SemaphoreType.DMA,
        pltpu.SemaphoreType.DMA,
        pltpu.SemaphoreType.DMA,
        pltpu.SemaphoreType.DMA,
    ],
)
def _gather_kernel(power_hbm, edge_hbm, vals_hbm,
                   table, src0, src1, val0, val1, si0, si1, so0, so1):
    ci = lax.axis_index("c")
    si = lax.axis_index("s")
    wid = ci * NS + si
    base0 = wid * EPW

    srcs = (src0, src1)
    vals = (val0, val1)
    sin = (si0, si1)
    sout = (so0, so1)

    pltpu.sync_copy(power_hbm, table)

    for b in range(2):
        pltpu.async_copy(
            edge_hbm.at[pl.ds(pl.multiple_of(base0 + b * CHUNK, 8), CHUNK)],
            srcs[b], sin[b])

    def group_body(g, carry):
        for b in range(2):
            @pl.when(g > 0)
            def _():
                pltpu.make_async_copy(
                    vals_hbm.at[pl.ds(0, CHUNK)], vals[b], sout[b]).wait()
            pltpu.make_async_copy(
                edge_hbm.at[pl.ds(0, CHUNK)], srcs[b], sin[b]).wait()

            def vec_body(i, c):
                for u in range(UNROLL):
                    off = pl.multiple_of((i * UNROLL + u) * 16, 16)
                    idx = srcs[b][pl.ds(off, 16)]
                    vals[b][pl.ds(off, 16)] = plsc.load_gather(table, [idx])
                return c
            lax.fori_loop(0, VPC // UNROLL, vec_body, 0)

            out_base = pl.multiple_of(base0 + (2 * g + b) * CHUNK, 8)
            pltpu.async_copy(vals[b], vals_hbm.at[pl.ds(out_base, CHUNK)],
                             sout[b])

            @pl.when(g < NGROUPS - 1)
            def _():
                nxt = pl.multiple_of(base0 + (2 * (g + 1) + b) * CHUNK, 8)
                pltpu.async_copy(edge_hbm.at[pl.ds(nxt, CHUNK)],
                                 srcs[b], sin[b])
        return carry

    lax.fori_loop(0, NGROUPS, group_body, 0)

    for b in range(2):
        pltpu.make_async_copy(
            vals_hbm.at[pl.ds(0, CHUNK)], vals[b], sout[b]).wait()


@functools.partial(
    pl.kernel,
    out_type=jax.ShapeDtypeStruct((NW, NPAD), jnp.float32),
    mesh=_mesh,
    compiler_params=pltpu.CompilerParams(needs_layout_passes=False),
    scratch_types=[
        pltpu.VMEM((NPAD,), jnp.float32),
        pltpu.VMEM((CHUNK,), jnp.int32),
        pltpu.VMEM((CHUNK,), jnp.int32),
        pltpu.VMEM((CHUNK,), jnp.float32),
        pltpu.VMEM((CHUNK,), jnp.float32),
        pltpu.SemaphoreType.DMA,
        pltpu.SemaphoreType.DMA,
    ],
)
def _scatter_kernel(edge_hbm, vals_hbm, part_hbm,
                    acc, dst0, dst1, val0, val1, si0, si1):
    ci = lax.axis_index("c")
    si = lax.axis_index("s")
    wid = ci * NS + si
    base0 = wid * EPW

    dsts = (dst0, dst1)
    vals = (val0, val1)
    sin = (si0, si1)

    for b in range(2):
        off = pl.multiple_of(base0 + b * CHUNK, 8)
        pltpu.async_copy(edge_hbm.at[pl.ds(N_EDGES + off, CHUNK)],
                         dsts[b], sin[b])
        pltpu.async_copy(vals_hbm.at[pl.ds(off, CHUNK)], vals[b], sin[b])

    def zero_body(i, carry):
        for u in range(8):
            off = pl.multiple_of((i * 8 + u) * 16, 16)
            acc[pl.ds(off, 16)] = jnp.zeros((16,), jnp.float32)
        return carry
    lax.fori_loop(0, NPAD // 128, zero_body, 0)

    def group_body(g, carry):
        for b in range(2):
            pltpu.make_async_copy(
                edge_hbm.at[pl.ds(0, CHUNK)], dsts[b], sin[b]).wait()
            pltpu.make_async_copy(
                vals_hbm.at[pl.ds(0, CHUNK)], vals[b], sin[b]).wait()

            def vec_body(i, c):
                for u in range(UNROLL):
                    off = pl.multiple_of((i * UNROLL + u) * 16, 16)
                    idx = dsts[b][pl.ds(off, 16)]
                    x = vals[b][pl.ds(off, 16)]
                    plsc.addupdate_scatter(acc, [idx], x)
                return c
            lax.fori_loop(0, VPC // UNROLL, vec_body, 0)

            @pl.when(g < NGROUPS - 1)
            def _():
                nxt = pl.multiple_of(base0 + (2 * (g + 1) + b) * CHUNK, 8)
                pltpu.async_copy(edge_hbm.at[pl.ds(N_EDGES + nxt, CHUNK)],
                                 dsts[b], sin[b])
                pltpu.async_copy(vals_hbm.at[pl.ds(nxt, CHUNK)],
                                 vals[b], sin[b])
        return carry

    lax.fori_loop(0, NGROUPS, group_body, 0)

    pltpu.sync_copy(acc, part_hbm.at[wid])


def _combine_body(ts_ref, p_ref, o_ref):
    o_ref[...] = jnp.sum(p_ref[...], axis=0) * ts_ref[0]


_combine = pl.pallas_call(
    _combine_body,
    grid=(7,),
    out_shape=jax.ShapeDtypeStruct((NPAD // 128, 128), jnp.float32),
    in_specs=[
        pl.BlockSpec(memory_space=pltpu.SMEM),
        pl.BlockSpec((NW, NPAD // 128 // 7, 128), lambda i: (0, i, 0)),
    ],
    out_specs=pl.BlockSpec((NPAD // 128 // 7, 128), lambda i: (i, 0)),
)


@jax.jit
def kernel(power, time_step, edge_index):
    edges = edge_index.astype(jnp.int32).reshape(-1)
    vals = _gather_kernel(power, edges)
    partials = _scatter_kernel(edges, vals)
    heat_pad = _combine(time_step, partials.reshape(NW, NPAD // 128, 128))
    return heat_pad.reshape(-1)[:N_NODES]
